# baseline (speedup 1.0000x reference)
import jax
import jax.numpy as jnp
import numpy as np
from jax import lax
from jax.experimental import pallas as pl
from jax.experimental.pallas import tpu as pltpu

B = 128
A = 100000
RG = 8
K = B // RG
TC_ = 256
NT = (A + TC_ - 1) // TC_

LOGP = np.float32(np.log(np.float64(np.float32(1.0 / A))))

_KS1 = np.uint32(42)
_KS2 = np.uint32(42 ^ 0x1BD11BDA)
_ROT_A = (13, 15, 26, 6)
_ROT_B = (17, 29, 16, 24)


def _rounds(x0, x1, rots):
    for d in rots:
        x0 = x0 + x1
        x1 = ((x1 << np.uint32(d)) | (x1 >> np.uint32(32 - d))) ^ x0
    return x0, x1


def _threefry_bits(x1):
    x0 = x1
    x1 = ((x1 << np.uint32(13)) | (x1 >> np.uint32(19))) ^ x0
    x0, x1 = _rounds(x0, x1, _ROT_A[1:])
    x0, x1 = x0 + _KS1, x1 + _KS2 + np.uint32(1)
    x0, x1 = _rounds(x0, x1, _ROT_B)
    x0, x1 = x0 + _KS2, x1 + np.uint32(2)
    x0, x1 = _rounds(x0, x1, _ROT_A)
    x0, x1 = x0, x1 + _KS1 + np.uint32(3)
    x0, x1 = _rounds(x0, x1, _ROT_B)
    x0, x1 = x0 + _KS1, x1 + _KS2 + np.uint32(4)
    x0, x1 = _rounds(x0, x1, _ROT_A)
    x0, x1 = x0 + _KS2, x1 + np.uint32(5)
    return x0 ^ x1


def _sample_kernel(actions_ref, logprob_ref, cbuf, sems):
    k = pl.program_id(0)

    @pl.when(k == 0)
    def _fill_buf():
        cbuf[...] = jnp.full((RG, A), LOGP, dtype=jnp.float32)

    @pl.when(k >= 2)
    def _drain():
        pltpu.make_async_copy(
            cbuf, logprob_ref.at[pl.ds((k - 2) * RG, RG), :],
            sems.at[k % 2]).wait()

    pltpu.make_async_copy(
        cbuf, logprob_ref.at[pl.ds(k * RG, RG), :], sems.at[k % 2]).start()

    colbase = jax.lax.broadcasted_iota(jnp.int32, (RG, TC_), 1)
    rowlin = (jax.lax.broadcasted_iota(jnp.int32, (RG, TC_), 0)
              + k * RG) * A

    def body(t, carry):
        mval, mcol = carry
        col = jnp.minimum(colbase + t * TC_, A - 1)
        x1 = (rowlin + col).astype(jnp.uint32) + _KS1
        m = (_threefry_bits(x1) >> np.uint32(9)).astype(jnp.int32)
        better = m > mval
        mval = jnp.maximum(m, mval)
        mcol = jnp.where(better, col, mcol)
        return mval, mcol

    mval, mcol = lax.fori_loop(
        0, NT, body,
        (jnp.full((RG, TC_), -1, jnp.int32),
         jnp.zeros((RG, TC_), jnp.int32)),
        unroll=2)

    bmax = jnp.max(mval, axis=1, keepdims=True)
    cand = jnp.where(mval == bmax, mcol, jnp.int32(2**31 - 1))
    actions_ref[...] = jnp.min(cand, axis=1, keepdims=True)

    @pl.when(k == K - 1)
    def _final_drain():
        for kk in (K - 2, K - 1):
            pltpu.make_async_copy(
                cbuf, logprob_ref.at[pl.ds(kk * RG, RG), :],
                sems.at[kk % 2]).wait()


@jax.jit
def _run():
    actions2d, logprob = pl.pallas_call(
        _sample_kernel,
        grid=(K,),
        out_specs=[
            pl.BlockSpec((RG, 1), lambda k: (k, 0)),
            pl.BlockSpec(memory_space=pl.ANY),
        ],
        out_shape=[
            jax.ShapeDtypeStruct((B, 1), jnp.int32),
            jax.ShapeDtypeStruct((B, A), jnp.float32),
        ],
        scratch_shapes=[
            pltpu.VMEM((RG, A), jnp.float32),
            pltpu.SemaphoreType.DMA((2,)),
        ],
    )()
    return actions2d.reshape(B), logprob


def kernel(state):
    del state
    return _run()


# speedup vs baseline: 1.2965x; 1.2965x over previous
import jax
import jax.numpy as jnp
import numpy as np
from jax import lax
from jax.experimental import pallas as pl
from jax.experimental.pallas import tpu as pltpu

B = 128
A = 100000
RG = 8
K = B // RG
TC_ = 256
NT = (A + TC_ - 1) // TC_

LOGP = np.float32(np.log(np.float64(np.float32(1.0 / A))))

_KS1 = np.uint32(42)
_KS2 = np.uint32(42 ^ 0x1BD11BDA)
_ROT_A = (13, 15, 26, 6)
_ROT_B = (17, 29, 16, 24)


def _rounds(x0, x1, rots):
    for d in rots:
        x0 = x0 + x1
        x1 = ((x1 << np.uint32(d)) | (x1 >> np.uint32(32 - d))) ^ x0
    return x0, x1


def _threefry_bits(x1):
    x0 = x1
    x1 = ((x1 << np.uint32(13)) | (x1 >> np.uint32(19))) ^ x0
    x0, x1 = _rounds(x0, x1, _ROT_A[1:])
    x0, x1 = x0 + _KS1, x1 + _KS2 + np.uint32(1)
    x0, x1 = _rounds(x0, x1, _ROT_B)
    x0, x1 = x0 + _KS2, x1 + np.uint32(2)
    x0, x1 = _rounds(x0, x1, _ROT_A)
    x0, x1 = x0, x1 + _KS1 + np.uint32(3)
    x0, x1 = _rounds(x0, x1, _ROT_B)
    x0, x1 = x0 + _KS1, x1 + _KS2 + np.uint32(4)
    x0, x1 = _rounds(x0, x1, _ROT_A)
    x0, x1 = x0 + _KS2, x1 + np.uint32(5)
    return x0 ^ x1


def _sample_kernel(actions_ref, logprob_ref, cbuf, sems):
    k = pl.program_id(0)

    @pl.when(k == 0)
    def _fill_buf():
        cbuf[...] = jnp.full((RG, A), LOGP, dtype=jnp.float32)

    @pl.when(k >= 2)
    def _drain():
        pltpu.make_async_copy(
            cbuf, logprob_ref.at[pl.ds((k - 2) * RG, RG), :],
            sems.at[k % 2]).wait()

    pltpu.make_async_copy(
        cbuf, logprob_ref.at[pl.ds(k * RG, RG), :], sems.at[k % 2]).start()

    colbase = jax.lax.broadcasted_iota(jnp.int32, (RG, TC_), 1)
    rowlin = (jax.lax.broadcasted_iota(jnp.int32, (RG, TC_), 0)
              + k * RG) * A

    def body(t, carry):
        mval, mcol = carry
        col = jnp.minimum(colbase + t * TC_, A - 1)
        x1 = (rowlin + col).astype(jnp.uint32) + _KS1
        m = (_threefry_bits(x1) >> np.uint32(9)).astype(jnp.int32)
        better = m > mval
        mval = jnp.maximum(m, mval)
        mcol = jnp.where(better, col, mcol)
        return mval, mcol

    mval, mcol = lax.fori_loop(
        0, NT, body,
        (jnp.full((RG, TC_), -1, jnp.int32),
         jnp.zeros((RG, TC_), jnp.int32)),
        unroll=8)

    bmax = jnp.max(mval, axis=1, keepdims=True)
    cand = jnp.where(mval == bmax, mcol, jnp.int32(2**31 - 1))
    actions_ref[...] = jnp.min(cand, axis=1, keepdims=True)

    @pl.when(k == K - 1)
    def _final_drain():
        for kk in (K - 2, K - 1):
            pltpu.make_async_copy(
                cbuf, logprob_ref.at[pl.ds(kk * RG, RG), :],
                sems.at[kk % 2]).wait()


@jax.jit
def _run():
    actions2d, logprob = pl.pallas_call(
        _sample_kernel,
        grid=(K,),
        out_specs=[
            pl.BlockSpec((RG, 1), lambda k: (k, 0)),
            pl.BlockSpec(memory_space=pl.ANY),
        ],
        out_shape=[
            jax.ShapeDtypeStruct((B, 1), jnp.int32),
            jax.ShapeDtypeStruct((B, A), jnp.float32),
        ],
        scratch_shapes=[
            pltpu.VMEM((RG, A), jnp.float32),
            pltpu.SemaphoreType.DMA((2,)),
        ],
    )()
    return actions2d.reshape(B), logprob


def kernel(state):
    del state
    return _run()


# speedup vs baseline: 1.3245x; 1.0216x over previous
import jax
import jax.numpy as jnp
import numpy as np
from jax import lax
from jax.experimental import pallas as pl
from jax.experimental.pallas import tpu as pltpu

B = 128
A = 100000
RG = 8
K = B // RG
TC_ = 256
NT = (A + TC_ - 1) // TC_

LOGP = np.float32(np.log(np.float64(np.float32(1.0 / A))))

_KS1 = np.uint32(42)
_KS2 = np.uint32(42 ^ 0x1BD11BDA)
_ROT_A = (13, 15, 26, 6)
_ROT_B = (17, 29, 16, 24)


def _rounds(x0, x1, rots):
    for d in rots:
        x0 = x0 + x1
        x1 = ((x1 << np.uint32(d)) | (x1 >> np.uint32(32 - d))) ^ x0
    return x0, x1


def _threefry_bits(x1):
    x0 = x1
    x1 = ((x1 << np.uint32(13)) | (x1 >> np.uint32(19))) ^ x0
    x0, x1 = _rounds(x0, x1, _ROT_A[1:])
    x0, x1 = x0 + _KS1, x1 + _KS2 + np.uint32(1)
    x0, x1 = _rounds(x0, x1, _ROT_B)
    x0, x1 = x0 + _KS2, x1 + np.uint32(2)
    x0, x1 = _rounds(x0, x1, _ROT_A)
    x0, x1 = x0, x1 + _KS1 + np.uint32(3)
    x0, x1 = _rounds(x0, x1, _ROT_B)
    x0, x1 = x0 + _KS1, x1 + _KS2 + np.uint32(4)
    x0, x1 = _rounds(x0, x1, _ROT_A)
    x0, x1 = x0 + _KS2, x1 + np.uint32(5)
    return x0 ^ x1


def _sample_kernel(actions_ref, logprob_ref, cbuf, sems):
    k = pl.program_id(0)

    @pl.when(k == 0)
    def _fill_buf():
        cbuf[...] = jnp.full((RG, A), LOGP, dtype=jnp.float32)

    @pl.when(k >= 2)
    def _drain():
        pltpu.make_async_copy(
            cbuf, logprob_ref.at[pl.ds((k - 2) * RG, RG), :],
            sems.at[k % 2]).wait()

    pltpu.make_async_copy(
        cbuf, logprob_ref.at[pl.ds(k * RG, RG), :], sems.at[k % 2]).start()

    colbase = jax.lax.broadcasted_iota(jnp.int32, (RG, TC_), 1)
    rowlin = (jax.lax.broadcasted_iota(jnp.int32, (RG, TC_), 0)
              + k * RG) * A

    def body(t, carry):
        mval, mcol = carry
        col = jnp.minimum(colbase + t * TC_, A - 1)
        x1 = (rowlin + col).astype(jnp.uint32) + _KS1
        m = (_threefry_bits(x1) >> np.uint32(9)).astype(jnp.int32)
        better = m > mval
        mval = jnp.maximum(m, mval)
        mcol = jnp.where(better, col, mcol)
        return mval, mcol

    mval, mcol = lax.fori_loop(
        0, NT, body,
        (jnp.full((RG, TC_), -1, jnp.int32),
         jnp.zeros((RG, TC_), jnp.int32)),
        unroll=16)

    bmax = jnp.max(mval, axis=1, keepdims=True)
    cand = jnp.where(mval == bmax, mcol, jnp.int32(2**31 - 1))
    actions_ref[...] = jnp.min(cand, axis=1, keepdims=True)

    @pl.when(k == K - 1)
    def _final_drain():
        for kk in (K - 2, K - 1):
            pltpu.make_async_copy(
                cbuf, logprob_ref.at[pl.ds(kk * RG, RG), :],
                sems.at[kk % 2]).wait()


@jax.jit
def _run():
    actions2d, logprob = pl.pallas_call(
        _sample_kernel,
        grid=(K,),
        out_specs=[
            pl.BlockSpec((RG, 1), lambda k: (k, 0)),
            pl.BlockSpec(memory_space=pl.ANY),
        ],
        out_shape=[
            jax.ShapeDtypeStruct((B, 1), jnp.int32),
            jax.ShapeDtypeStruct((B, A), jnp.float32),
        ],
        scratch_shapes=[
            pltpu.VMEM((RG, A), jnp.float32),
            pltpu.SemaphoreType.DMA((2,)),
        ],
    )()
    return actions2d.reshape(B), logprob


def kernel(state):
    del state
    return _run()


# speedup vs baseline: 1.3385x; 1.0106x over previous
import jax
import jax.numpy as jnp
import numpy as np
from jax import lax
from jax.experimental import pallas as pl
from jax.experimental.pallas import tpu as pltpu

B = 128
A = 100000
RG = 8
K = B // RG
TC_ = 256
NT = (A + TC_ - 1) // TC_

LOGP = np.float32(np.log(np.float64(np.float32(1.0 / A))))

_KS1 = np.uint32(42)
_KS2 = np.uint32(42 ^ 0x1BD11BDA)
_ROT_A = (13, 15, 26, 6)
_ROT_B = (17, 29, 16, 24)


def _rounds(x0, x1, rots):
    for d in rots:
        x0 = x0 + x1
        x1 = ((x1 << np.uint32(d)) | (x1 >> np.uint32(32 - d))) ^ x0
    return x0, x1


def _threefry_bits(x1):
    x0 = x1
    x1 = ((x1 << np.uint32(13)) | (x1 >> np.uint32(19))) ^ x0
    x0, x1 = _rounds(x0, x1, _ROT_A[1:])
    x0, x1 = x0 + _KS1, x1 + _KS2 + np.uint32(1)
    x0, x1 = _rounds(x0, x1, _ROT_B)
    x0, x1 = x0 + _KS2, x1 + np.uint32(2)
    x0, x1 = _rounds(x0, x1, _ROT_A)
    x0, x1 = x0, x1 + _KS1 + np.uint32(3)
    x0, x1 = _rounds(x0, x1, _ROT_B)
    x0, x1 = x0 + _KS1, x1 + _KS2 + np.uint32(4)
    x0, x1 = _rounds(x0, x1, _ROT_A)
    x0, x1 = x0 + _KS2, x1 + np.uint32(5)
    return x0 ^ x1


def _sample_kernel(actions_ref, logprob_ref, cbuf, sems):
    k = pl.program_id(0)

    @pl.when(k == 0)
    def _fill_buf():
        cbuf[...] = jnp.full((RG, A), LOGP, dtype=jnp.float32)

    @pl.when(k >= 2)
    def _drain():
        pltpu.make_async_copy(
            cbuf, logprob_ref.at[pl.ds((k - 2) * RG, RG), :],
            sems.at[k % 2]).wait()

    pltpu.make_async_copy(
        cbuf, logprob_ref.at[pl.ds(k * RG, RG), :], sems.at[k % 2]).start()

    colbase = jax.lax.broadcasted_iota(jnp.int32, (RG, TC_), 1)
    rowlin = (jax.lax.broadcasted_iota(jnp.int32, (RG, TC_), 0)
              + k * RG) * A

    def body(t, carry):
        mval, mcol = carry
        col = jnp.minimum(colbase + t * TC_, A - 1)
        x1 = (rowlin + col).astype(jnp.uint32) + _KS1
        m = (_threefry_bits(x1) >> np.uint32(9)).astype(jnp.int32)
        better = m > mval
        mval = jnp.maximum(m, mval)
        mcol = jnp.where(better, col, mcol)
        return mval, mcol

    mval, mcol = lax.fori_loop(
        0, NT, body,
        (jnp.full((RG, TC_), -1, jnp.int32),
         jnp.zeros((RG, TC_), jnp.int32)),
        unroll=32)

    bmax = jnp.max(mval, axis=1, keepdims=True)
    cand = jnp.where(mval == bmax, mcol, jnp.int32(2**31 - 1))
    actions_ref[...] = jnp.min(cand, axis=1, keepdims=True)

    @pl.when(k == K - 1)
    def _final_drain():
        for kk in (K - 2, K - 1):
            pltpu.make_async_copy(
                cbuf, logprob_ref.at[pl.ds(kk * RG, RG), :],
                sems.at[kk % 2]).wait()


@jax.jit
def _run():
    actions2d, logprob = pl.pallas_call(
        _sample_kernel,
        grid=(K,),
        out_specs=[
            pl.BlockSpec((RG, 1), lambda k: (k, 0)),
            pl.BlockSpec(memory_space=pl.ANY),
        ],
        out_shape=[
            jax.ShapeDtypeStruct((B, 1), jnp.int32),
            jax.ShapeDtypeStruct((B, A), jnp.float32),
        ],
        scratch_shapes=[
            pltpu.VMEM((RG, A), jnp.float32),
            pltpu.SemaphoreType.DMA((2,)),
        ],
    )()
    return actions2d.reshape(B), logprob


def kernel(state):
    del state
    return _run()


# speedup vs baseline: 1.3409x; 1.0018x over previous
import jax
import jax.numpy as jnp
import numpy as np
from jax import lax
from jax.experimental import pallas as pl
from jax.experimental.pallas import tpu as pltpu

B = 128
A = 100000
RG = 8
K = B // RG
TC_ = 512
NT = (A + TC_ - 1) // TC_

LOGP = np.float32(np.log(np.float64(np.float32(1.0 / A))))

_KS1 = np.uint32(42)
_KS2 = np.uint32(42 ^ 0x1BD11BDA)
_ROT_A = (13, 15, 26, 6)
_ROT_B = (17, 29, 16, 24)


def _rounds(x0, x1, rots):
    for d in rots:
        x0 = x0 + x1
        x1 = ((x1 << np.uint32(d)) | (x1 >> np.uint32(32 - d))) ^ x0
    return x0, x1


def _threefry_bits(x1):
    x0 = x1
    x1 = ((x1 << np.uint32(13)) | (x1 >> np.uint32(19))) ^ x0
    x0, x1 = _rounds(x0, x1, _ROT_A[1:])
    x0, x1 = x0 + _KS1, x1 + _KS2 + np.uint32(1)
    x0, x1 = _rounds(x0, x1, _ROT_B)
    x0, x1 = x0 + _KS2, x1 + np.uint32(2)
    x0, x1 = _rounds(x0, x1, _ROT_A)
    x0, x1 = x0, x1 + _KS1 + np.uint32(3)
    x0, x1 = _rounds(x0, x1, _ROT_B)
    x0, x1 = x0 + _KS1, x1 + _KS2 + np.uint32(4)
    x0, x1 = _rounds(x0, x1, _ROT_A)
    x0, x1 = x0 + _KS2, x1 + np.uint32(5)
    return x0 ^ x1


def _sample_kernel(actions_ref, logprob_ref, cbuf, sems):
    k = pl.program_id(0)

    @pl.when(k == 0)
    def _fill_buf():
        cbuf[...] = jnp.full((RG, A), LOGP, dtype=jnp.float32)

    @pl.when(k >= 2)
    def _drain():
        pltpu.make_async_copy(
            cbuf, logprob_ref.at[pl.ds((k - 2) * RG, RG), :],
            sems.at[k % 2]).wait()

    pltpu.make_async_copy(
        cbuf, logprob_ref.at[pl.ds(k * RG, RG), :], sems.at[k % 2]).start()

    colbase = jax.lax.broadcasted_iota(jnp.int32, (RG, TC_), 1)
    rowlin = (jax.lax.broadcasted_iota(jnp.int32, (RG, TC_), 0)
              + k * RG) * A

    def body(t, carry):
        mval, mcol = carry
        col = jnp.minimum(colbase + t * TC_, A - 1)
        x1 = (rowlin + col).astype(jnp.uint32) + _KS1
        m = (_threefry_bits(x1) >> np.uint32(9)).astype(jnp.int32)
        better = m > mval
        mval = jnp.maximum(m, mval)
        mcol = jnp.where(better, col, mcol)
        return mval, mcol

    mval, mcol = lax.fori_loop(
        0, NT, body,
        (jnp.full((RG, TC_), -1, jnp.int32),
         jnp.zeros((RG, TC_), jnp.int32)),
        unroll=16)

    bmax = jnp.max(mval, axis=1, keepdims=True)
    cand = jnp.where(mval == bmax, mcol, jnp.int32(2**31 - 1))
    actions_ref[...] = jnp.min(cand, axis=1, keepdims=True)

    @pl.when(k == K - 1)
    def _final_drain():
        for kk in (K - 2, K - 1):
            pltpu.make_async_copy(
                cbuf, logprob_ref.at[pl.ds(kk * RG, RG), :],
                sems.at[kk % 2]).wait()


@jax.jit
def _run():
    actions2d, logprob = pl.pallas_call(
        _sample_kernel,
        grid=(K,),
        out_specs=[
            pl.BlockSpec((RG, 1), lambda k: (k, 0)),
            pl.BlockSpec(memory_space=pl.ANY),
        ],
        out_shape=[
            jax.ShapeDtypeStruct((B, 1), jnp.int32),
            jax.ShapeDtypeStruct((B, A), jnp.float32),
        ],
        scratch_shapes=[
            pltpu.VMEM((RG, A), jnp.float32),
            pltpu.SemaphoreType.DMA((2,)),
        ],
    )()
    return actions2d.reshape(B), logprob


def kernel(state):
    del state
    return _run()


# speedup vs baseline: 1.3418x; 1.0007x over previous
import jax
import jax.numpy as jnp
import numpy as np
from jax import lax
from jax.experimental import pallas as pl
from jax.experimental.pallas import tpu as pltpu

B = 128
A = 100000
RG = 8
K = B // RG
TC_ = 512
NT = (A + TC_ - 1) // TC_

LOGP = np.float32(np.log(np.float64(np.float32(1.0 / A))))

_KS1 = np.uint32(42)
_KS2 = np.uint32(42 ^ 0x1BD11BDA)
_ROT_A = (13, 15, 26, 6)
_ROT_B = (17, 29, 16, 24)


def _rounds(x0, x1, rots):
    for d in rots:
        x0 = x0 + x1
        x1 = ((x1 << np.uint32(d)) | (x1 >> np.uint32(32 - d))) ^ x0
    return x0, x1


def _threefry_bits(x1):
    x0 = x1
    x1 = ((x1 << np.uint32(13)) | (x1 >> np.uint32(19))) ^ x0
    x0, x1 = _rounds(x0, x1, _ROT_A[1:])
    x0, x1 = x0 + _KS1, x1 + _KS2 + np.uint32(1)
    x0, x1 = _rounds(x0, x1, _ROT_B)
    x0, x1 = x0 + _KS2, x1 + np.uint32(2)
    x0, x1 = _rounds(x0, x1, _ROT_A)
    x0, x1 = x0, x1 + _KS1 + np.uint32(3)
    x0, x1 = _rounds(x0, x1, _ROT_B)
    x0, x1 = x0 + _KS1, x1 + _KS2 + np.uint32(4)
    x0, x1 = _rounds(x0, x1, _ROT_A)
    x0, x1 = x0 + _KS2, x1 + np.uint32(5)
    return x0 ^ x1


def _sample_kernel(actions_ref, logprob_ref, cbuf, sems):
    k = pl.program_id(0)

    @pl.when(k == 0)
    def _fill_buf():
        cbuf[...] = jnp.full((RG, A), LOGP, dtype=jnp.float32)

    @pl.when(k >= 4)
    def _drain():
        pltpu.make_async_copy(
            cbuf, logprob_ref.at[pl.ds((k - 4) * RG, RG), :],
            sems.at[k % 4]).wait()

    pltpu.make_async_copy(
        cbuf, logprob_ref.at[pl.ds(k * RG, RG), :], sems.at[k % 4]).start()

    colbase = jax.lax.broadcasted_iota(jnp.int32, (RG, TC_), 1)
    rowlin = (jax.lax.broadcasted_iota(jnp.int32, (RG, TC_), 0)
              + k * RG) * A

    def body(t, carry):
        mval, mcol = carry
        col = jnp.minimum(colbase + t * TC_, A - 1)
        x1 = (rowlin + col).astype(jnp.uint32) + _KS1
        m = (_threefry_bits(x1) >> np.uint32(9)).astype(jnp.int32)
        better = m > mval
        mval = jnp.maximum(m, mval)
        mcol = jnp.where(better, col, mcol)
        return mval, mcol

    mval, mcol = lax.fori_loop(
        0, NT, body,
        (jnp.full((RG, TC_), -1, jnp.int32),
         jnp.zeros((RG, TC_), jnp.int32)),
        unroll=16)

    bmax = jnp.max(mval, axis=1, keepdims=True)
    cand = jnp.where(mval == bmax, mcol, jnp.int32(2**31 - 1))
    actions_ref[...] = jnp.min(cand, axis=1, keepdims=True)

    @pl.when(k == K - 1)
    def _final_drain():
        for kk in (K - 4, K - 3, K - 2, K - 1):
            pltpu.make_async_copy(
                cbuf, logprob_ref.at[pl.ds(kk * RG, RG), :],
                sems.at[kk % 4]).wait()


@jax.jit
def _run():
    actions2d, logprob = pl.pallas_call(
        _sample_kernel,
        grid=(K,),
        out_specs=[
            pl.BlockSpec((RG, 1), lambda k: (k, 0)),
            pl.BlockSpec(memory_space=pl.ANY),
        ],
        out_shape=[
            jax.ShapeDtypeStruct((B, 1), jnp.int32),
            jax.ShapeDtypeStruct((B, A), jnp.float32),
        ],
        scratch_shapes=[
            pltpu.VMEM((RG, A), jnp.float32),
            pltpu.SemaphoreType.DMA((4,)),
        ],
    )()
    return actions2d.reshape(B), logprob


def kernel(state):
    del state
    return _run()
